# Initial kernel scaffold; baseline (speedup 1.0000x reference)
import jax
import jax.numpy as jnp
from jax.experimental import pallas as pl


def kernel(pred, gt, mask):
    raise NotImplementedError("write your pallas kernel here")


# speedup vs baseline: 18.2230x; 18.2230x over previous
import functools

import jax
import jax.numpy as jnp
from jax import lax
from jax.experimental import pallas as pl

_N, _H, _W = 8, 512, 512
_NEG_RATIO = 3.0
_EPS = 1e-08


def _per_sample_kernel(pred_ref, gt_ref, mask_ref, out_ref):
    p = pred_ref[0]
    g = gt_ref[0]
    m = mask_ref[0]

    pos = g * m
    neg = (1.0 - g) * m
    pos_count = jnp.sum(pos)
    neg_count_raw = jnp.sum(neg)
    neg_count = jnp.minimum(neg_count_raw, pos_count * _NEG_RATIO)
    k = jnp.floor(neg_count).astype(jnp.int32)

    log_p = jnp.maximum(jnp.log(p), -100.0)
    log_1mp = jnp.maximum(jnp.log(1.0 - p), -100.0)
    loss = -(g * log_p + (1.0 - g) * log_1mp)
    pos_loss_sum = jnp.sum(loss * pos)

    v = loss * neg
    v_bits = lax.bitcast_convert_type(v, jnp.int32)

    def body(i, prefix):
        b = 30 - i
        cand = prefix | (1 << b)
        cnt = jnp.sum((v_bits >= cand).astype(jnp.int32))
        return jnp.where(cnt >= k, cand, prefix)

    t_bits = lax.fori_loop(0, 31, body, jnp.int32(0))
    t = lax.bitcast_convert_type(t_bits, jnp.float32)

    gt_mask = v_bits > t_bits
    cnt_gt = jnp.sum(gt_mask.astype(jnp.int32))
    sum_gt = jnp.sum(jnp.where(gt_mask, v, 0.0))
    top_neg_sum = jnp.where(
        k > 0, sum_gt + (k - cnt_gt).astype(jnp.float32) * t, 0.0
    )

    per_sample = (pos_loss_sum + top_neg_sum) / (pos_count + neg_count + _EPS)
    out_ref[:, :, :] = per_sample[None, None, None]


@jax.jit
def kernel(pred, gt, mask):
    p = pred.reshape(_N, _H, _W)
    per_sample = pl.pallas_call(
        _per_sample_kernel,
        grid=(_N,),
        in_specs=[
            pl.BlockSpec((1, _H, _W), lambda i: (i, 0, 0)),
            pl.BlockSpec((1, _H, _W), lambda i: (i, 0, 0)),
            pl.BlockSpec((1, _H, _W), lambda i: (i, 0, 0)),
        ],
        out_specs=pl.BlockSpec((1, 1, 1), lambda i: (i, 0, 0)),
        out_shape=jax.ShapeDtypeStruct((_N, 1, 1), jnp.float32),
    )(p, gt, mask)
    return jnp.sum(per_sample) / _N


# speedup vs baseline: 28.2589x; 1.5507x over previous
import functools

import jax
import jax.numpy as jnp
from jax import lax
from jax.experimental import pallas as pl

_N, _H, _W = 8, 512, 512
_NEG_RATIO = 3.0
_EPS = 1e-08


def _per_sample_kernel(pred_ref, gt_ref, mask_ref, out_ref):
    p = pred_ref[0]
    g = gt_ref[0]
    m = mask_ref[0]

    pos = g * m
    neg = (1.0 - g) * m
    pos_count = jnp.sum(pos)
    neg_count_raw = jnp.sum(neg)
    neg_count = jnp.minimum(neg_count_raw, pos_count * _NEG_RATIO)
    k = jnp.floor(neg_count).astype(jnp.int32)

    log_p = jnp.maximum(jnp.log(p), -100.0)
    log_1mp = jnp.maximum(jnp.log(1.0 - p), -100.0)
    loss = -(g * log_p + (1.0 - g) * log_1mp)
    pos_loss_sum = jnp.sum(loss * pos)

    v = loss * neg
    v_bits = lax.bitcast_convert_type(v, jnp.int32)

    def body(i, prefix):
        b = 30 - i
        cand = prefix | (1 << b)
        cnt = jnp.sum((v_bits >= cand).astype(jnp.int32))
        return jnp.where(cnt >= k, cand, prefix)

    t_bits = lax.fori_loop(0, 18, body, jnp.int32(0))
    t = lax.bitcast_convert_type(t_bits, jnp.float32)

    gt_mask = v_bits > t_bits
    cnt_gt = jnp.sum(gt_mask.astype(jnp.int32))
    sum_gt = jnp.sum(jnp.where(gt_mask, v, 0.0))
    top_neg_sum = jnp.where(
        k > 0, sum_gt + (k - cnt_gt).astype(jnp.float32) * t, 0.0
    )

    per_sample = (pos_loss_sum + top_neg_sum) / (pos_count + neg_count + _EPS)
    out_ref[:, :, :] = per_sample[None, None, None]


@jax.jit
def kernel(pred, gt, mask):
    p = pred.reshape(_N, _H, _W)
    per_sample = pl.pallas_call(
        _per_sample_kernel,
        grid=(_N,),
        in_specs=[
            pl.BlockSpec((1, _H, _W), lambda i: (i, 0, 0)),
            pl.BlockSpec((1, _H, _W), lambda i: (i, 0, 0)),
            pl.BlockSpec((1, _H, _W), lambda i: (i, 0, 0)),
        ],
        out_specs=pl.BlockSpec((1, 1, 1), lambda i: (i, 0, 0)),
        out_shape=jax.ShapeDtypeStruct((_N, 1, 1), jnp.float32),
    )(p, gt, mask)
    return jnp.sum(per_sample) / _N
